# Initial kernel scaffold; baseline (speedup 1.0000x reference)
import jax
import jax.numpy as jnp
from jax.experimental import pallas as pl


def kernel(u, t, triangulation, cell_centers, cell_local_vertex_pos, free_form_data, inv_mass, W, b):
    raise NotImplementedError("write your pallas kernel here")


# speedup vs baseline: 7.0825x; 7.0825x over previous
import jax
import jax.numpy as jnp
from jax import lax
from jax.experimental import pallas as pl
from jax.experimental.pallas import tpu as pltpu
from jax.experimental.pallas import tpu_sc as plsc

N_NODES = 50000
N_CELLS = 100000
D = 128

NUM_CORES = 2
NUM_SUBCORES = 16
NUM_TILES = NUM_CORES * NUM_SUBCORES

CELLS_PER_TILE = 3200
NC_PAD = NUM_TILES * CELLS_PER_TILE
STREAM_B = 128
STREAMS_PER_TILE = 3 * CELLS_PER_TILE // STREAM_B
TRI_ROWS_PER_TILE = 80
BIG_STREAMS = 24
BIG_CELLS = BIG_STREAMS * STREAM_B // 3
TAIL_STREAMS = 3
TAIL_CELLS = TAIL_STREAMS * STREAM_B // 3
NUM_BIG = 3


def _q_matmul_body(u_ref, w_ref, q_ref):
    q_ref[...] = jnp.dot(u_ref[...], w_ref[...],
                         preferred_element_type=jnp.float32)


def _compute_q(u2d, wcat):
    blk = 2000
    return pl.pallas_call(
        _q_matmul_body,
        grid=(N_NODES // blk,),
        in_specs=[
            pl.BlockSpec((blk, D), lambda i: (i, 0)),
            pl.BlockSpec((D, 16), lambda i: (0, 0)),
        ],
        out_specs=pl.BlockSpec((blk, 16), lambda i: (i, 0)),
        out_shape=jax.ShapeDtypeStruct((N_NODES, 16), jnp.float32),
    )(u2d, wcat)


def _base_body(w_ref, f_ref, c_ref, o_ref):
    o_ref[...] = jnp.dot(w_ref[...], f_ref[...],
                         preferred_element_type=jnp.float32) + c_ref[...]


def _compute_base(wft, feats_t, const):
    blk = NC_PAD // 8
    return pl.pallas_call(
        _base_body,
        grid=(NC_PAD // blk,),
        in_specs=[
            pl.BlockSpec((8, 8), lambda i: (0, 0)),
            pl.BlockSpec((8, blk), lambda i: (0, i)),
            pl.BlockSpec((8, 1), lambda i: (0, 0)),
        ],
        out_specs=pl.BlockSpec((8, blk), lambda i: (0, i)),
        out_shape=jax.ShapeDtypeStruct((8, NC_PAD), jnp.float32),
    )(wft, feats_t, const)


def _sc_body(q_hbm, tri_hbm, bf_hbm, zeros_hbm, out0_hbm, out1_hbm,
             idx_v, rows_v, bf_v, contrib_v, sem, accum_sh):
    core = lax.axis_index("c")
    sub = lax.axis_index("s")
    wid = core * NUM_SUBCORES + sub

    @pl.when(sub == 0)
    def _():
        pltpu.sync_copy(zeros_hbm, accum_sh)

    plsc.subcore_barrier()

    iota = lax.iota(jnp.int32, 16)
    three_iota = iota * 3
    lane_off = [jnp.full((16,), 3 * j + i, jnp.int32)
                for j in range(3) for i in range(3)]

    tri_row0 = wid * TRI_ROWS_PER_TILE
    cell0 = wid * CELLS_PER_TILE

    def do_chunk(row0, c0, nstreams, ncells):
        ncopy = (nstreams + 7) // 8 * 8
        pltpu.sync_copy(tri_hbm.at[pl.ds(row0, ncopy)],
                        idx_v.at[pl.ds(0, ncopy)])
        pltpu.sync_copy(bf_hbm.at[:, pl.ds(c0, ncells)],
                        bf_v.at[:, pl.ds(0, ncells)])

        descs = []
        for s in range(nstreams):
            descs.append(
                pltpu.async_copy(q_hbm.at[idx_v.at[s]],
                                 rows_v.at[pl.ds(s * STREAM_B, STREAM_B)],
                                 sem))
        for d in descs:
            d.wait()

        for g in range(ncells // 16):
            pair = [three_iota + (48 * g + k) for k in range(3)]
            p = [lax.shift_right_logical(pk, 7) for pk in pair]
            q = [lax.bitwise_and(pk, 127) for pk in pair]
            for i in range(3):
                s0 = plsc.load_gather(rows_v, [pair[0], lane_off[0 + i]])
                s1 = plsc.load_gather(rows_v, [pair[1], lane_off[3 + i]])
                s2 = plsc.load_gather(rows_v, [pair[2], lane_off[6 + i]])
                base_i = bf_v[i, pl.ds(g * 16, 16)]
                ffd_i = bf_v[3 + i, pl.ds(g * 16, 16)]
                contrib = ffd_i * (base_i + (s0 + s1) + s2)
                plsc.store_scatter(contrib_v, [p[i], q[i]], contrib)

        for s in range(nstreams):
            pltpu.sync_copy(contrib_v.at[s], accum_sh.at[idx_v.at[s]],
                            add=True)

    def big_body(ch, _):
        do_chunk(tri_row0 + ch * BIG_STREAMS, cell0 + ch * BIG_CELLS,
                 BIG_STREAMS, BIG_CELLS)
        return ()

    lax.fori_loop(0, NUM_BIG, big_body, ())
    do_chunk(tri_row0 + NUM_BIG * BIG_STREAMS, cell0 + NUM_BIG * BIG_CELLS,
             TAIL_STREAMS, TAIL_CELLS)

    plsc.subcore_barrier()

    @pl.when(jnp.logical_and(sub == 0, core == 0))
    def _():
        pltpu.sync_copy(accum_sh, out0_hbm)

    @pl.when(jnp.logical_and(sub == 0, core == 1))
    def _():
        pltpu.sync_copy(accum_sh, out1_hbm)


def _sc_scatter(q, tri_rows, bf, zeros):
    mesh = plsc.VectorSubcoreMesh(core_axis_name="c", subcore_axis_name="s")
    kern = pl.kernel(
        _sc_body,
        out_type=(jax.ShapeDtypeStruct((N_NODES,), jnp.float32),
                  jax.ShapeDtypeStruct((N_NODES,), jnp.float32)),
        mesh=mesh,
        compiler_params=pltpu.CompilerParams(needs_layout_passes=False,
                                             use_tc_tiling_on_sc=False),
        scratch_types=[
            pltpu.VMEM((BIG_STREAMS, STREAM_B), jnp.int32),
            pltpu.VMEM((BIG_STREAMS * STREAM_B, 16), jnp.float32),
            pltpu.VMEM((6, BIG_CELLS), jnp.float32),
            pltpu.VMEM((BIG_STREAMS, STREAM_B), jnp.float32),
            pltpu.SemaphoreType.DMA,
            pltpu.VMEM_SHARED((N_NODES,), jnp.float32),
        ],
    )
    return kern(q, tri_rows, bf, zeros)


def _combine_body(p0_ref, p1_ref, m_ref, o_ref):
    o_ref[...] = (p0_ref[...] + p1_ref[...]) * m_ref[...]


def _combine(p0, p1, inv_mass2d):
    return pl.pallas_call(
        _combine_body,
        out_shape=jax.ShapeDtypeStruct((1, N_NODES), jnp.float32),
    )(p0[None, :], p1[None, :], inv_mass2d)


@jax.jit
def kernel(u, t, triangulation, cell_centers, cell_local_vertex_pos,
           free_form_data, inv_mass, W, b):
    u2d = u[0]

    wv = W[9:].reshape(3, D, 3)
    wcat = jnp.transpose(wv, (1, 0, 2)).reshape(D, 9)
    wcat = jnp.pad(wcat, ((0, 0), (0, 7)))
    q = _compute_q(u2d, wcat)

    pad_c = NC_PAD - N_CELLS
    feats8 = jnp.concatenate(
        [cell_centers, cell_local_vertex_pos.reshape(N_CELLS, 6)], axis=1)
    feats8_t = jnp.pad(feats8, ((0, pad_c), (0, 0))).T
    wft = jnp.pad(W[1:9].T, ((0, 5), (0, 0)))
    const = jnp.pad(t[0, 0] * W[0] + b, (0, 5))[:, None]
    base_t = _compute_base(wft, feats8_t, const)[:3]

    ffd_t = jnp.pad(free_form_data, ((0, pad_c), (0, 0))).T
    bf = jnp.concatenate([base_t, ffd_t], axis=0)

    tri_rows = jnp.pad(triangulation, ((0, pad_c), (0, 0))).reshape(
        NUM_TILES, STREAMS_PER_TILE, STREAM_B)
    tri_rows = jnp.pad(tri_rows, ((0, 0), (0, TRI_ROWS_PER_TILE -
                                           STREAMS_PER_TILE), (0, 0)))
    tri_rows = tri_rows.reshape(NUM_TILES * TRI_ROWS_PER_TILE, STREAM_B)

    zeros = jnp.zeros((N_NODES,), jnp.float32)

    p0, p1 = _sc_scatter(q, tri_rows, bf, zeros)

    return _combine(p0, p1, inv_mass[None, :])
